# baseline (speedup 1.0000x reference)
import functools

import jax
import jax.numpy as jnp
from jax import lax
from jax.experimental import pallas as pl
from jax.experimental.pallas import tpu as pltpu
from jax.experimental.pallas import tpu_sc as plsc

NUM_USER = 10000
NUM_ITEM = 40000
NUM_NODES = NUM_USER + NUM_ITEM

NC = 2
NS = 16
NW = NC * NS
CW = 16
BLK = 1024

U_ROWS = 10112
I_ROWS = 40064
W_ROWS = 30080


def _make_seg(n_chunks, nb, acc_rows, tab_rows, cw):
    mesh = plsc.VectorSubcoreMesh(core_axis_name="c", subcore_axis_name="s")
    rpt = acc_rows // NS
    tpr = tab_rows // NS
    npair = nb // 2

    @functools.partial(
        pl.kernel,
        out_type=jax.ShapeDtypeStruct((NC, n_chunks, acc_rows, cw), jnp.float32),
        mesh=mesh,
        compiler_params=pltpu.CompilerParams(use_tc_tiling_on_sc=False),
        scratch_types=[
            pltpu.VMEM((BLK,), jnp.int32),
            pltpu.VMEM((BLK,), jnp.int32),
            pltpu.VMEM((BLK,), jnp.int32),
            pltpu.VMEM((BLK,), jnp.int32),
            pltpu.VMEM((BLK, cw), jnp.float32),
            pltpu.VMEM((BLK, cw), jnp.float32),
            pltpu.VMEM_SHARED((tab_rows, cw), jnp.float32),
            pltpu.VMEM_SHARED((acc_rows, cw), jnp.float32),
            pltpu.SemaphoreType.DMA,
            pltpu.SemaphoreType.DMA,
        ],
    )
    def seg_kernel(*refs):
        tables = refs[:n_chunks]
        src1_h, dst1_h, zeros_h, out_h = refs[n_chunks:n_chunks + 4]
        (src_a, dst_a, src_b, dst_b, rows_a, rows_b, tab_s, acc,
         sem_a, sem_b) = refs[n_chunks + 4:]
        c = lax.axis_index("c")
        s = lax.axis_index("s")
        base = (c * NS + s) * (nb * BLK)

        def fire(eb, src1, dst1, rows, sem, cc):
            pltpu.sync_copy(src1_h.at[pl.ds(eb, BLK)], src1)
            pltpu.sync_copy(dst1_h.at[pl.ds(eb, BLK)], dst1)
            pltpu.async_copy(tab_s.at[src1], rows, sem)

        def drain(rows, sem):
            pltpu.make_async_copy(zeros_h.at[pl.ds(0, BLK)], rows, sem).wait()

        def scatter(rows, dst1):
            pltpu.sync_copy(rows, acc.at[dst1], add=True)

        for cc in range(n_chunks):
            pltpu.sync_copy(tables[cc].at[pl.ds(s * tpr, tpr)],
                            tab_s.at[pl.ds(s * tpr, tpr)])
            pltpu.sync_copy(zeros_h.at[pl.ds(s * rpt, rpt)],
                            acc.at[pl.ds(s * rpt, rpt)])
            plsc.subcore_barrier()
            fire(base, src_a, dst_a, rows_a, sem_a, cc)

            def pair_body(b2, carry, cc=cc):
                eb0 = base + (2 * b2) * BLK
                fire(eb0 + BLK, src_b, dst_b, rows_b, sem_b, cc)
                drain(rows_a, sem_a)
                scatter(rows_a, dst_a)
                fire(eb0 + 2 * BLK, src_a, dst_a, rows_a, sem_a, cc)
                drain(rows_b, sem_b)
                scatter(rows_b, dst_b)
                return carry

            lax.fori_loop(0, npair, pair_body, 0)
            drain(rows_a, sem_a)
            plsc.subcore_barrier()
            pltpu.sync_copy(acc.at[pl.ds(s * rpt, rpt)],
                            out_h.at[c, cc, pl.ds(s * rpt, rpt)])
            plsc.subcore_barrier()

    return seg_kernel


def _make_gather9():
    mesh = plsc.VectorSubcoreMesh(core_axis_name="c", subcore_axis_name="s")
    per_w = 1024 // NW

    @functools.partial(
        pl.kernel,
        out_type=jax.ShapeDtypeStruct((9, 1024, 64), jnp.float32),
        mesh=mesh,
        compiler_params=pltpu.CompilerParams(use_tc_tiling_on_sc=False),
        scratch_types=[
            pltpu.VMEM((per_w,), jnp.int32),
            pltpu.VMEM((per_w, 64), jnp.float32),
            pltpu.SemaphoreType.DMA,
        ],
    )
    def gather_kernel(rep_t, rep_v, rep_a, users, poss, negs, out_h,
                      idx_v, rows_v, sem):
        c = lax.axis_index("c")
        s = lax.axis_index("s")
        base = (c * NS + s) * per_w
        k = 0
        for rep in (rep_t, rep_v, rep_a):
            for idxh in (users, poss, negs):
                pltpu.sync_copy(idxh.at[pl.ds(base, per_w)], idx_v)
                pltpu.async_copy(rep.at[idx_v], rows_v, sem).wait()
                pltpu.sync_copy(rows_v, out_h.at[k, pl.ds(base, per_w)])
                k += 1

    return gather_kernel


def _lrelu(v):
    return jnp.where(v >= 0, v, 0.01 * v)


R = 2000


def _mm_bias_body(f_ref, w_ref, b_ref, o_ref):
    o_ref[...] = (jnp.dot(f_ref[...], w_ref[...],
                          preferred_element_type=jnp.float32) + b_ref[...])


def _mm_bias(feat, w, b):
    n, f = feat.shape
    dout = w.shape[1]
    return pl.pallas_call(
        _mm_bias_body,
        grid=(n // R,),
        in_specs=[
            pl.BlockSpec((R, f), lambda i: (i, 0)),
            pl.BlockSpec((f, dout), lambda i: (0, 0)),
            pl.BlockSpec((1, dout), lambda i: (0, 0)),
        ],
        out_specs=pl.BlockSpec((R, dout), lambda i: (i, 0)),
        out_shape=jax.ShapeDtypeStruct((n, dout), jnp.float32),
    )(feat, w, b.reshape(1, -1))


def _t1_body(s_ref, c_ref, w_ref, b_ref, o_ref):
    den = jnp.maximum(c_ref[0] + c_ref[1], 1.0)
    tf = (s_ref[0] + s_ref[1]) / den
    o_ref[...] = (jnp.dot(tf, w_ref[...],
                          preferred_element_type=jnp.float32) + b_ref[...])


def _t1(hps, hpc, w, b):
    return pl.pallas_call(
        _t1_body,
        grid=(NUM_ITEM // R,),
        in_specs=[
            pl.BlockSpec((2, R, 128), lambda i: (0, i, 0)),
            pl.BlockSpec((2, R, 1), lambda i: (0, i, 0)),
            pl.BlockSpec((128, 128), lambda i: (0, 0)),
            pl.BlockSpec((1, 128), lambda i: (0, 0)),
        ],
        out_specs=pl.BlockSpec((R, 128), lambda i: (i, 0)),
        out_shape=jax.ShapeDtypeStruct((NUM_ITEM, 128), jnp.float32),
    )(hps, hpc, w, b.reshape(1, -1))


def _a2_body(x0_ref, cw_ref, x_ref, *m_refs):
    x0 = x0_ref[...]
    nrm = jnp.sqrt(jnp.sum(x0 * x0, axis=1, keepdims=True))
    x = x0 / jnp.maximum(nrm, 1e-12)
    x_ref[...] = x
    m = jnp.dot(x, cw_ref[...], preferred_element_type=jnp.float32)
    for cc in range(8):
        m_refs[cc][...] = m[:, CW * cc:CW * cc + CW]


def _a2(x0, conv1_w):
    outs = pl.pallas_call(
        _a2_body,
        grid=(NUM_NODES // R,),
        in_specs=[
            pl.BlockSpec((R, 128), lambda i: (i, 0)),
            pl.BlockSpec((128, 128), lambda i: (0, 0)),
        ],
        out_specs=[pl.BlockSpec((R, 128), lambda i: (i, 0))]
        + [pl.BlockSpec((R, CW), lambda i: (i, 0))] * 8,
        out_shape=[jax.ShapeDtypeStruct((NUM_NODES, 128), jnp.float32)]
        + [jax.ShapeDtypeStruct((NUM_NODES, CW), jnp.float32)] * 8,
    )(x0, conv1_w)
    return outs[0], outs[1:]


def _b_body(hp_ref, x_ref, id_ref, l1w_ref, l1b_ref, g1w_ref, g1b_ref,
            c2w_ref, x2_ref, *m_refs):
    h = _lrelu(hp_ref[0] + hp_ref[1])
    hg = jnp.dot(h, g1w_ref[...], preferred_element_type=jnp.float32)
    xh = _lrelu(jnp.dot(x_ref[...], l1w_ref[...],
                        preferred_element_type=jnp.float32)
                + l1b_ref[...]) + id_ref[...]
    x2 = _lrelu(hg + g1b_ref[...] + xh)
    x2_ref[...] = x2
    m = jnp.dot(x2, c2w_ref[...], preferred_element_type=jnp.float32)
    for cc in range(4):
        m_refs[cc][...] = m[:, CW * cc:CW * cc + CW]


def _b_stage(hp, x, id_emb, p):
    outs = pl.pallas_call(
        _b_body,
        grid=(NUM_NODES // R,),
        in_specs=[
            pl.BlockSpec((2, R, 128), lambda i: (0, i, 0)),
            pl.BlockSpec((R, 128), lambda i: (i, 0)),
            pl.BlockSpec((R, 64), lambda i: (i, 0)),
            pl.BlockSpec((128, 64), lambda i: (0, 0)),
            pl.BlockSpec((1, 64), lambda i: (0, 0)),
            pl.BlockSpec((128, 64), lambda i: (0, 0)),
            pl.BlockSpec((1, 64), lambda i: (0, 0)),
            pl.BlockSpec((64, 64), lambda i: (0, 0)),
        ],
        out_specs=[pl.BlockSpec((R, 64), lambda i: (i, 0))]
        + [pl.BlockSpec((R, CW), lambda i: (i, 0))] * 4,
        out_shape=[jax.ShapeDtypeStruct((NUM_NODES, 64), jnp.float32)]
        + [jax.ShapeDtypeStruct((NUM_NODES, CW), jnp.float32)] * 4,
    )(hp, x, id_emb, p['lin1_w'], p['lin1_b'].reshape(1, -1),
      p['g1_w'], p['g1_b'].reshape(1, -1), p['conv2_w'])
    return outs[0], outs[1:]


def _c_body(hp_ref, x2_ref, id_ref, l2w_ref, l2b_ref, g2w_ref, g2b_ref,
            rep_ref):
    h = _lrelu(hp_ref[0] + hp_ref[1])
    hg = jnp.dot(h, g2w_ref[...], preferred_element_type=jnp.float32)
    xh = _lrelu(jnp.dot(x2_ref[...], l2w_ref[...],
                        preferred_element_type=jnp.float32)
                + l2b_ref[...]) + id_ref[...]
    rep_ref[...] = _lrelu(hg + g2b_ref[...] + xh)


def _c_stage(hp, x2, id_emb, p):
    return pl.pallas_call(
        _c_body,
        grid=(NUM_NODES // R,),
        in_specs=[
            pl.BlockSpec((2, R, 64), lambda i: (0, i, 0)),
            pl.BlockSpec((R, 64), lambda i: (i, 0)),
            pl.BlockSpec((R, 64), lambda i: (i, 0)),
            pl.BlockSpec((64, 64), lambda i: (0, 0)),
            pl.BlockSpec((1, 64), lambda i: (0, 0)),
            pl.BlockSpec((64, 64), lambda i: (0, 0)),
            pl.BlockSpec((1, 64), lambda i: (0, 0)),
        ],
        out_specs=pl.BlockSpec((R, 64), lambda i: (i, 0)),
        out_shape=jax.ShapeDtypeStruct((NUM_NODES, 64), jnp.float32),
    )(hp, x2, id_emb, p['lin2_w'], p['lin2_b'].reshape(1, -1),
      p['g2_w'], p['g2_b'].reshape(1, -1))


def _score_body(g_ref, o_ref):
    gt_u, gt_p, gt_n = g_ref[0], g_ref[1], g_ref[2]
    gv_u, gv_p, gv_n = g_ref[3], g_ref[4], g_ref[5]
    ga_u, ga_p, ga_n = g_ref[6], g_ref[7], g_ref[8]
    pre_pos = jnp.sum(gt_u * gt_p, axis=1)
    pre_neg = jnp.sum(gt_u * gt_n, axis=1)
    pu = (gt_u + gv_u + ga_u) / 3.0
    pp = (gt_p + gv_p + ga_p) / 3.0
    pn = (gt_n + gv_n + ga_n) / 3.0
    post_pos = jnp.sum(pu * pp, axis=1)
    post_neg = jnp.sum(pu * pn, axis=1)
    o_ref[0, :] = post_pos * (1.0 / (1.0 + jnp.exp(-pre_pos)))
    o_ref[1, :] = post_neg * (1.0 / (1.0 + jnp.exp(-pre_neg)))
    o_ref[2, :] = pre_pos
    o_ref[3, :] = pre_neg


def _pad_idx(idx, pad_val, total):
    out = jnp.full((total,), pad_val, jnp.int32)
    return lax.dynamic_update_slice(out, idx.astype(jnp.int32), (0,))


def _pad_tab(t, rows):
    return jnp.pad(t, ((0, rows - t.shape[0]), (0, 0)))


def kernel(v_feat, a_feat, words_tensor, edge_index, id_embedding, word_emb,
           v_params, a_params, t_params, user_nodes, pos_item_nodes,
           neg_item_nodes):
    E = edge_index.shape[1]
    NI = E // 2
    W = words_tensor.shape[1]
    unit = NW * BLK
    nb_e = 2 * -(-NI // (2 * unit))
    nb_w = 2 * -(-W // (2 * unit))
    e_pad = nb_e * unit + BLK
    w_pad = nb_w * unit + BLK

    src, dst = edge_index[0], edge_index[1]
    src_u = _pad_idx(src[:NI], 0, e_pad)
    dst_i = _pad_idx(dst[:NI] - NUM_USER, NUM_ITEM, e_pad)
    src_i = _pad_idx(src[NI:] - NUM_USER, 0, e_pad)
    dst_u = _pad_idx(dst[NI:], NUM_USER, e_pad)
    wgat = _pad_idx(words_tensor[1], 0, w_pad)
    wsct = _pad_idx(words_tensor[0], NUM_ITEM, w_pad)

    zeros16 = jnp.zeros((I_ROWS, CW), jnp.float32)
    zeros8 = jnp.zeros((I_ROWS, 8), jnp.float32)

    seg_ui8 = _make_seg(8, nb_e, I_ROWS, U_ROWS, CW)
    seg_iu8 = _make_seg(8, nb_e, U_ROWS, I_ROWS, CW)
    seg_ui4 = _make_seg(4, nb_e, I_ROWS, U_ROWS, CW)
    seg_iu4 = _make_seg(4, nb_e, U_ROWS, I_ROWS, CW)
    seg_w = _make_seg(17, nb_w, I_ROWS, W_ROWS, 8)

    def seg_edges(m_chunks, seg_ui, seg_iu):
        tabs_u = [_pad_tab(mc[:NUM_USER], U_ROWS) for mc in m_chunks]
        tabs_i = [_pad_tab(mc[NUM_USER:], I_ROWS) for mc in m_chunks]
        hp_i = seg_ui(*tabs_u, src_u, dst_i, zeros16)
        hp_u = seg_iu(*tabs_i, src_i, dst_u, zeros16)
        hp = jnp.concatenate(
            [hp_u[:, :, :NUM_USER], hp_i[:, :, :NUM_ITEM]], axis=2)
        n_chunks = hp.shape[1]
        return hp.transpose(0, 2, 1, 3).reshape(2, NUM_NODES, n_chunks * CW)

    def gcn(p, temp):
        x0 = jnp.concatenate([p['preference'], temp], axis=0)
        x, m1c = _a2(x0, p['conv1_w'])
        hp1 = seg_edges(m1c, seg_ui8, seg_iu8)
        x2, m2c = _b_stage(hp1, x, id_embedding, p)
        hp2 = seg_edges(m2c, seg_ui4, seg_iu4)
        return _c_stage(hp2, x2, id_embedding, p)

    temp_v = _mm_bias(v_feat, v_params['mlp_w'], v_params['mlp_b'])
    temp_a = _mm_bias(a_feat, a_params['mlp_w'], a_params['mlp_b'])
    rep_v = gcn(v_params, temp_v)
    rep_a = gcn(a_params, temp_a)

    wtabs = [_pad_tab(word_emb[:, 8 * cc:8 * cc + 8], W_ROWS)
             for cc in range(16)]
    ones_tab = jnp.zeros((W_ROWS, 8), jnp.float32).at[:, 0].set(1.0)
    hpw = seg_w(*wtabs, ones_tab, wgat, wsct, zeros8)
    hpw = hpw[:, :, :NUM_ITEM].transpose(0, 2, 1, 3).reshape(2, NUM_ITEM, 136)
    temp_t = _t1(hpw[:, :, :128], hpw[:, :, 128:129],
                 t_params['mlp_w'], t_params['mlp_b'])
    rep_t = gcn(t_params, temp_t)

    g9 = _make_gather9()(
        rep_t, rep_v, rep_a,
        user_nodes.astype(jnp.int32), pos_item_nodes.astype(jnp.int32),
        neg_item_nodes.astype(jnp.int32))
    o = pl.pallas_call(
        _score_body,
        out_shape=jax.ShapeDtypeStruct((4, 1024), jnp.float32),
    )(g9)
    return (o[0], o[1], o[2], o[3])


# speedup vs baseline: 1.1894x; 1.1894x over previous
import functools

import jax
import jax.numpy as jnp
from jax import lax
from jax.experimental import pallas as pl
from jax.experimental.pallas import tpu as pltpu
from jax.experimental.pallas import tpu_sc as plsc

NUM_USER = 10000
NUM_ITEM = 40000
NUM_NODES = NUM_USER + NUM_ITEM

NC = 2
NS = 16
NW = NC * NS
CW = 16
BLK = 1024

U_ROWS = 10112
I_ROWS = 40064
W_ROWS = 30080


def _make_seg(n_chunks, nb, acc_rows, tab_rows, cw):
    mesh = plsc.VectorSubcoreMesh(core_axis_name="c", subcore_axis_name="s")
    rpt = acc_rows // NS
    tpr = tab_rows // NS
    npair = nb // 2

    @functools.partial(
        pl.kernel,
        out_type=jax.ShapeDtypeStruct((NC, n_chunks, acc_rows, cw), jnp.float32),
        mesh=mesh,
        compiler_params=pltpu.CompilerParams(use_tc_tiling_on_sc=False),
        scratch_types=[
            pltpu.VMEM((BLK,), jnp.int32),
            pltpu.VMEM((BLK,), jnp.int32),
            pltpu.VMEM((BLK,), jnp.int32),
            pltpu.VMEM((BLK,), jnp.int32),
            pltpu.VMEM((BLK, cw), jnp.float32),
            pltpu.VMEM((BLK, cw), jnp.float32),
            pltpu.VMEM_SHARED((tab_rows, cw), jnp.float32),
            pltpu.VMEM_SHARED((acc_rows, cw), jnp.float32),
            pltpu.SemaphoreType.DMA,
            pltpu.SemaphoreType.DMA,
        ],
    )
    def seg_kernel(*refs):
        tables = refs[:n_chunks]
        src1_h, dst1_h, zeros_h, out_h = refs[n_chunks:n_chunks + 4]
        (src_a, dst_a, src_b, dst_b, rows_a, rows_b, tab_s, acc,
         sem_a, sem_b) = refs[n_chunks + 4:]
        c = lax.axis_index("c")
        s = lax.axis_index("s")
        base = (c * NS + s) * (nb * BLK)

        def fire(eb, src1, dst1, rows, sem, cc):
            pltpu.sync_copy(src1_h.at[pl.ds(eb, BLK)], src1)
            pltpu.sync_copy(dst1_h.at[pl.ds(eb, BLK)], dst1)
            pltpu.async_copy(tab_s.at[src1], rows, sem)

        def drain(rows, sem):
            pltpu.make_async_copy(zeros_h.at[pl.ds(0, BLK)], rows, sem).wait()

        def scatter(rows, dst1):
            pltpu.sync_copy(rows, acc.at[dst1], add=True)

        for cc in range(n_chunks):
            pltpu.sync_copy(tables[cc].at[pl.ds(s * tpr, tpr)],
                            tab_s.at[pl.ds(s * tpr, tpr)])
            pltpu.sync_copy(zeros_h.at[pl.ds(s * rpt, rpt)],
                            acc.at[pl.ds(s * rpt, rpt)])
            plsc.subcore_barrier()
            fire(base, src_a, dst_a, rows_a, sem_a, cc)

            def pair_body(b2, carry, cc=cc):
                eb0 = base + (2 * b2) * BLK
                fire(eb0 + BLK, src_b, dst_b, rows_b, sem_b, cc)
                drain(rows_a, sem_a)
                scatter(rows_a, dst_a)
                fire(eb0 + 2 * BLK, src_a, dst_a, rows_a, sem_a, cc)
                drain(rows_b, sem_b)
                scatter(rows_b, dst_b)
                return carry

            lax.fori_loop(0, npair, pair_body, 0)
            drain(rows_a, sem_a)
            plsc.subcore_barrier()
            pltpu.sync_copy(acc.at[pl.ds(s * rpt, rpt)],
                            out_h.at[c, cc, pl.ds(s * rpt, rpt)])
            plsc.subcore_barrier()

    return seg_kernel


def _make_gather9():
    mesh = plsc.VectorSubcoreMesh(core_axis_name="c", subcore_axis_name="s")
    per_w = 1024 // NW

    @functools.partial(
        pl.kernel,
        out_type=jax.ShapeDtypeStruct((9, 1024, 64), jnp.float32),
        mesh=mesh,
        compiler_params=pltpu.CompilerParams(use_tc_tiling_on_sc=False),
        scratch_types=[
            pltpu.VMEM((per_w,), jnp.int32),
            pltpu.VMEM((per_w, 64), jnp.float32),
            pltpu.SemaphoreType.DMA,
        ],
    )
    def gather_kernel(rep_t, rep_v, rep_a, users, poss, negs, out_h,
                      idx_v, rows_v, sem):
        c = lax.axis_index("c")
        s = lax.axis_index("s")
        base = (c * NS + s) * per_w
        k = 0
        for rep in (rep_t, rep_v, rep_a):
            for idxh in (users, poss, negs):
                pltpu.sync_copy(idxh.at[pl.ds(base, per_w)], idx_v)
                pltpu.async_copy(rep.at[idx_v], rows_v, sem).wait()
                pltpu.sync_copy(rows_v, out_h.at[k, pl.ds(base, per_w)])
                k += 1

    return gather_kernel


def _lrelu(v):
    return jnp.where(v >= 0, v, 0.01 * v)


R = 2000


def _mm_bias_body(f_ref, w_ref, b_ref, o_ref):
    o_ref[...] = (jnp.dot(f_ref[...], w_ref[...],
                          preferred_element_type=jnp.float32) + b_ref[...])


def _mm_bias(feat, w, b):
    n, f = feat.shape
    dout = w.shape[1]
    return pl.pallas_call(
        _mm_bias_body,
        grid=(n // R,),
        in_specs=[
            pl.BlockSpec((R, f), lambda i: (i, 0)),
            pl.BlockSpec((f, dout), lambda i: (0, 0)),
            pl.BlockSpec((1, dout), lambda i: (0, 0)),
        ],
        out_specs=pl.BlockSpec((R, dout), lambda i: (i, 0)),
        out_shape=jax.ShapeDtypeStruct((n, dout), jnp.float32),
    )(feat, w, b.reshape(1, -1))


def _t1_body(s_ref, c_ref, w_ref, b_ref, o_ref):
    den = jnp.maximum(c_ref[0] + c_ref[1], 1.0)
    tf = (s_ref[0] + s_ref[1]) / den
    o_ref[...] = (jnp.dot(tf, w_ref[...],
                          preferred_element_type=jnp.float32) + b_ref[...])


def _t1(hps, hpc, w, b):
    return pl.pallas_call(
        _t1_body,
        grid=(NUM_ITEM // R,),
        in_specs=[
            pl.BlockSpec((2, R, 128), lambda i: (0, i, 0)),
            pl.BlockSpec((2, R, 1), lambda i: (0, i, 0)),
            pl.BlockSpec((128, 128), lambda i: (0, 0)),
            pl.BlockSpec((1, 128), lambda i: (0, 0)),
        ],
        out_specs=pl.BlockSpec((R, 128), lambda i: (i, 0)),
        out_shape=jax.ShapeDtypeStruct((NUM_ITEM, 128), jnp.float32),
    )(hps, hpc, w, b.reshape(1, -1))


def _a2_body(x0_ref, cw_ref, x_ref, *m_refs):
    x0 = x0_ref[...]
    nrm = jnp.sqrt(jnp.sum(x0 * x0, axis=1, keepdims=True))
    x = x0 / jnp.maximum(nrm, 1e-12)
    x_ref[...] = x
    m = jnp.dot(x, cw_ref[...], preferred_element_type=jnp.float32)
    for cc in range(8):
        m_refs[cc][...] = m[:, CW * cc:CW * cc + CW]


def _a2(x0, conv1_w):
    outs = pl.pallas_call(
        _a2_body,
        grid=(NUM_NODES // R,),
        in_specs=[
            pl.BlockSpec((R, 128), lambda i: (i, 0)),
            pl.BlockSpec((128, 128), lambda i: (0, 0)),
        ],
        out_specs=[pl.BlockSpec((R, 128), lambda i: (i, 0))]
        + [pl.BlockSpec((R, CW), lambda i: (i, 0))] * 8,
        out_shape=[jax.ShapeDtypeStruct((NUM_NODES, 128), jnp.float32)]
        + [jax.ShapeDtypeStruct((NUM_NODES, CW), jnp.float32)] * 8,
    )(x0, conv1_w)
    return outs[0], outs[1:]


def _b_body(hp_ref, x_ref, id_ref, l1w_ref, l1b_ref, g1w_ref, g1b_ref,
            c2w_ref, x2_ref, *m_refs):
    h = _lrelu(hp_ref[0] + hp_ref[1])
    hg = jnp.dot(h, g1w_ref[...], preferred_element_type=jnp.float32)
    xh = _lrelu(jnp.dot(x_ref[...], l1w_ref[...],
                        preferred_element_type=jnp.float32)
                + l1b_ref[...]) + id_ref[...]
    x2 = _lrelu(hg + g1b_ref[...] + xh)
    x2_ref[...] = x2
    m = jnp.dot(x2, c2w_ref[...], preferred_element_type=jnp.float32)
    for cc in range(4):
        m_refs[cc][...] = m[:, CW * cc:CW * cc + CW]


def _b_stage(hp, x, id_emb, p):
    outs = pl.pallas_call(
        _b_body,
        grid=(NUM_NODES // R,),
        in_specs=[
            pl.BlockSpec((2, R, 128), lambda i: (0, i, 0)),
            pl.BlockSpec((R, 128), lambda i: (i, 0)),
            pl.BlockSpec((R, 64), lambda i: (i, 0)),
            pl.BlockSpec((128, 64), lambda i: (0, 0)),
            pl.BlockSpec((1, 64), lambda i: (0, 0)),
            pl.BlockSpec((128, 64), lambda i: (0, 0)),
            pl.BlockSpec((1, 64), lambda i: (0, 0)),
            pl.BlockSpec((64, 64), lambda i: (0, 0)),
        ],
        out_specs=[pl.BlockSpec((R, 64), lambda i: (i, 0))]
        + [pl.BlockSpec((R, CW), lambda i: (i, 0))] * 4,
        out_shape=[jax.ShapeDtypeStruct((NUM_NODES, 64), jnp.float32)]
        + [jax.ShapeDtypeStruct((NUM_NODES, CW), jnp.float32)] * 4,
    )(hp, x, id_emb, p['lin1_w'], p['lin1_b'].reshape(1, -1),
      p['g1_w'], p['g1_b'].reshape(1, -1), p['conv2_w'])
    return outs[0], outs[1:]


def _c_body(hp_ref, x2_ref, id_ref, l2w_ref, l2b_ref, g2w_ref, g2b_ref,
            rep_ref):
    h = _lrelu(hp_ref[0] + hp_ref[1])
    hg = jnp.dot(h, g2w_ref[...], preferred_element_type=jnp.float32)
    xh = _lrelu(jnp.dot(x2_ref[...], l2w_ref[...],
                        preferred_element_type=jnp.float32)
                + l2b_ref[...]) + id_ref[...]
    rep_ref[...] = _lrelu(hg + g2b_ref[...] + xh)


def _c_stage(hp, x2, id_emb, p):
    return pl.pallas_call(
        _c_body,
        grid=(NUM_NODES // R,),
        in_specs=[
            pl.BlockSpec((2, R, 64), lambda i: (0, i, 0)),
            pl.BlockSpec((R, 64), lambda i: (i, 0)),
            pl.BlockSpec((R, 64), lambda i: (i, 0)),
            pl.BlockSpec((64, 64), lambda i: (0, 0)),
            pl.BlockSpec((1, 64), lambda i: (0, 0)),
            pl.BlockSpec((64, 64), lambda i: (0, 0)),
            pl.BlockSpec((1, 64), lambda i: (0, 0)),
        ],
        out_specs=pl.BlockSpec((R, 64), lambda i: (i, 0)),
        out_shape=jax.ShapeDtypeStruct((NUM_NODES, 64), jnp.float32),
    )(hp, x2, id_emb, p['lin2_w'], p['lin2_b'].reshape(1, -1),
      p['g2_w'], p['g2_b'].reshape(1, -1))


def _score_body(g_ref, o_ref):
    gt_u, gt_p, gt_n = g_ref[0], g_ref[1], g_ref[2]
    gv_u, gv_p, gv_n = g_ref[3], g_ref[4], g_ref[5]
    ga_u, ga_p, ga_n = g_ref[6], g_ref[7], g_ref[8]
    pre_pos = jnp.sum(gt_u * gt_p, axis=1)
    pre_neg = jnp.sum(gt_u * gt_n, axis=1)
    pu = (gt_u + gv_u + ga_u) / 3.0
    pp = (gt_p + gv_p + ga_p) / 3.0
    pn = (gt_n + gv_n + ga_n) / 3.0
    post_pos = jnp.sum(pu * pp, axis=1)
    post_neg = jnp.sum(pu * pn, axis=1)
    o_ref[0, :] = post_pos * (1.0 / (1.0 + jnp.exp(-pre_pos)))
    o_ref[1, :] = post_neg * (1.0 / (1.0 + jnp.exp(-pre_neg)))
    o_ref[2, :] = pre_pos
    o_ref[3, :] = pre_neg


def _pad_idx(idx, pad_val, total, spread=1):
    pads = pad_val + (jnp.arange(total, dtype=jnp.int32) % spread)
    return lax.dynamic_update_slice(pads, idx.astype(jnp.int32), (0,))


def _pad_tab(t, rows):
    return jnp.pad(t, ((0, rows - t.shape[0]), (0, 0)))


def kernel(v_feat, a_feat, words_tensor, edge_index, id_embedding, word_emb,
           v_params, a_params, t_params, user_nodes, pos_item_nodes,
           neg_item_nodes):
    E = edge_index.shape[1]
    NI = E // 2
    W = words_tensor.shape[1]
    unit = NW * BLK
    nb_e = 2 * -(-NI // (2 * unit))
    nb_w = 2 * -(-W // (2 * unit))
    e_pad = nb_e * unit + BLK
    w_pad = nb_w * unit + BLK

    src, dst = edge_index[0], edge_index[1]
    src_u = _pad_idx(src[:NI], 0, e_pad, 8192)
    dst_i = _pad_idx(dst[:NI] - NUM_USER, NUM_ITEM, e_pad, 64)
    src_i = _pad_idx(src[NI:] - NUM_USER, 0, e_pad, 8192)
    dst_u = _pad_idx(dst[NI:], NUM_USER, e_pad, 112)
    wgat = _pad_idx(words_tensor[1], 0, w_pad, 8192)
    wsct = _pad_idx(words_tensor[0], NUM_ITEM, w_pad, 64)

    zeros16 = jnp.zeros((I_ROWS, CW), jnp.float32)
    zeros8 = jnp.zeros((I_ROWS, 8), jnp.float32)

    seg_ui8 = _make_seg(8, nb_e, I_ROWS, U_ROWS, CW)
    seg_iu8 = _make_seg(8, nb_e, U_ROWS, I_ROWS, CW)
    seg_ui4 = _make_seg(4, nb_e, I_ROWS, U_ROWS, CW)
    seg_iu4 = _make_seg(4, nb_e, U_ROWS, I_ROWS, CW)
    seg_w = _make_seg(17, nb_w, I_ROWS, W_ROWS, 8)

    def seg_edges(m_chunks, seg_ui, seg_iu):
        tabs_u = [_pad_tab(mc[:NUM_USER], U_ROWS) for mc in m_chunks]
        tabs_i = [_pad_tab(mc[NUM_USER:], I_ROWS) for mc in m_chunks]
        hp_i = seg_ui(*tabs_u, src_u, dst_i, zeros16)
        hp_u = seg_iu(*tabs_i, src_i, dst_u, zeros16)
        hp = jnp.concatenate(
            [hp_u[:, :, :NUM_USER], hp_i[:, :, :NUM_ITEM]], axis=2)
        n_chunks = hp.shape[1]
        return hp.transpose(0, 2, 1, 3).reshape(2, NUM_NODES, n_chunks * CW)

    def gcn(p, temp):
        x0 = jnp.concatenate([p['preference'], temp], axis=0)
        x, m1c = _a2(x0, p['conv1_w'])
        hp1 = seg_edges(m1c, seg_ui8, seg_iu8)
        x2, m2c = _b_stage(hp1, x, id_embedding, p)
        hp2 = seg_edges(m2c, seg_ui4, seg_iu4)
        return _c_stage(hp2, x2, id_embedding, p)

    temp_v = _mm_bias(v_feat, v_params['mlp_w'], v_params['mlp_b'])
    temp_a = _mm_bias(a_feat, a_params['mlp_w'], a_params['mlp_b'])
    rep_v = gcn(v_params, temp_v)
    rep_a = gcn(a_params, temp_a)

    wtabs = [_pad_tab(word_emb[:, 8 * cc:8 * cc + 8], W_ROWS)
             for cc in range(16)]
    ones_tab = jnp.zeros((W_ROWS, 8), jnp.float32).at[:, 0].set(1.0)
    hpw = seg_w(*wtabs, ones_tab, wgat, wsct, zeros8)
    hpw = hpw[:, :, :NUM_ITEM].transpose(0, 2, 1, 3).reshape(2, NUM_ITEM, 136)
    temp_t = _t1(hpw[:, :, :128], hpw[:, :, 128:129],
                 t_params['mlp_w'], t_params['mlp_b'])
    rep_t = gcn(t_params, temp_t)

    g9 = _make_gather9()(
        rep_t, rep_v, rep_a,
        user_nodes.astype(jnp.int32), pos_item_nodes.astype(jnp.int32),
        neg_item_nodes.astype(jnp.int32))
    o = pl.pallas_call(
        _score_body,
        out_shape=jax.ShapeDtypeStruct((4, 1024), jnp.float32),
    )(g9)
    return (o[0], o[1], o[2], o[3])


# speedup vs baseline: 1.8789x; 1.5797x over previous
import functools

import jax
import jax.numpy as jnp
from jax import lax
from jax.experimental import pallas as pl
from jax.experimental.pallas import tpu as pltpu
from jax.experimental.pallas import tpu_sc as plsc

NUM_USER = 10000
NUM_ITEM = 40000
NUM_NODES = NUM_USER + NUM_ITEM

NC = 2
NS = 16
NW = NC * NS
CW = 16
BLK = 1024

U_ROWS = 10112
I_ROWS = 40064
W_ROWS = 30080


def _make_seg(n_chunks, nb, acc_rows, tab_rows, cw):
    mesh = plsc.VectorSubcoreMesh(core_axis_name="c", subcore_axis_name="s")
    rpt = acc_rows // NS
    tpr = tab_rows // NS
    npair = nb // 2

    @functools.partial(
        pl.kernel,
        out_type=jax.ShapeDtypeStruct((NC, acc_rows, n_chunks * cw), jnp.float32),
        mesh=mesh,
        compiler_params=pltpu.CompilerParams(use_tc_tiling_on_sc=False),
        scratch_types=[
            pltpu.VMEM((BLK,), jnp.int32),
            pltpu.VMEM((BLK,), jnp.int32),
            pltpu.VMEM((BLK,), jnp.int32),
            pltpu.VMEM((BLK,), jnp.int32),
            pltpu.VMEM((BLK, cw), jnp.float32),
            pltpu.VMEM((BLK, cw), jnp.float32),
            pltpu.VMEM_SHARED((tab_rows, cw), jnp.float32),
            pltpu.VMEM_SHARED((acc_rows, cw), jnp.float32),
            pltpu.SemaphoreType.DMA,
            pltpu.SemaphoreType.DMA,
        ],
    )
    def seg_kernel(*refs):
        tables = refs[:n_chunks]
        src1_h, dst1_h, zeros_h, out_h = refs[n_chunks:n_chunks + 4]
        (src_a, dst_a, src_b, dst_b, rows_a, rows_b, tab_s, acc,
         sem_a, sem_b) = refs[n_chunks + 4:]
        c = lax.axis_index("c")
        s = lax.axis_index("s")
        base = (c * NS + s) * (nb * BLK)

        def fire(eb, src1, dst1, rows, sem, cc):
            pltpu.sync_copy(src1_h.at[pl.ds(eb, BLK)], src1)
            pltpu.sync_copy(dst1_h.at[pl.ds(eb, BLK)], dst1)
            pltpu.async_copy(tab_s.at[src1], rows, sem)

        def drain(rows, sem):
            pltpu.make_async_copy(zeros_h.at[pl.ds(0, BLK)], rows, sem).wait()

        def scatter(rows, dst1):
            pltpu.sync_copy(rows, acc.at[dst1], add=True)

        for cc in range(n_chunks):
            pltpu.sync_copy(tables[cc].at[pl.ds(s * tpr, tpr)],
                            tab_s.at[pl.ds(s * tpr, tpr)])
            pltpu.sync_copy(zeros_h.at[pl.ds(s * rpt, rpt)],
                            acc.at[pl.ds(s * rpt, rpt)])
            plsc.subcore_barrier()
            fire(base, src_a, dst_a, rows_a, sem_a, cc)

            def pair_body(b2, carry, cc=cc):
                eb0 = base + (2 * b2) * BLK
                fire(eb0 + BLK, src_b, dst_b, rows_b, sem_b, cc)
                drain(rows_a, sem_a)
                scatter(rows_a, dst_a)
                fire(eb0 + 2 * BLK, src_a, dst_a, rows_a, sem_a, cc)
                drain(rows_b, sem_b)
                scatter(rows_b, dst_b)
                return carry

            lax.fori_loop(0, npair, pair_body, 0)
            drain(rows_a, sem_a)
            plsc.subcore_barrier()
            pltpu.sync_copy(acc.at[pl.ds(s * rpt, rpt)],
                            out_h.at[c, pl.ds(s * rpt, rpt),
                                     pl.ds(cc * cw, cw)])
            plsc.subcore_barrier()

    return seg_kernel


def _make_gather9():
    mesh = plsc.VectorSubcoreMesh(core_axis_name="c", subcore_axis_name="s")
    per_w = 1024 // NW

    @functools.partial(
        pl.kernel,
        out_type=jax.ShapeDtypeStruct((9, 1024, 64), jnp.float32),
        mesh=mesh,
        compiler_params=pltpu.CompilerParams(use_tc_tiling_on_sc=False),
        scratch_types=[
            pltpu.VMEM((per_w,), jnp.int32),
            pltpu.VMEM((per_w, 64), jnp.float32),
            pltpu.SemaphoreType.DMA,
        ],
    )
    def gather_kernel(rep_t, rep_v, rep_a, users, poss, negs, out_h,
                      idx_v, rows_v, sem):
        c = lax.axis_index("c")
        s = lax.axis_index("s")
        base = (c * NS + s) * per_w
        k = 0
        for rep in (rep_t, rep_v, rep_a):
            for idxh in (users, poss, negs):
                pltpu.sync_copy(idxh.at[pl.ds(base, per_w)], idx_v)
                pltpu.async_copy(rep.at[idx_v], rows_v, sem).wait()
                pltpu.sync_copy(rows_v, out_h.at[k, pl.ds(base, per_w)])
                k += 1

    return gather_kernel


def _lrelu(v):
    return jnp.where(v >= 0, v, 0.01 * v)


R = 2000


def _mm_bias_body(f_ref, w_ref, b_ref, o_ref):
    o_ref[...] = (jnp.dot(f_ref[...], w_ref[...],
                          preferred_element_type=jnp.float32) + b_ref[...])


def _mm_bias(feat, w, b):
    n, f = feat.shape
    dout = w.shape[1]
    return pl.pallas_call(
        _mm_bias_body,
        grid=(n // R,),
        in_specs=[
            pl.BlockSpec((R, f), lambda i: (i, 0)),
            pl.BlockSpec((f, dout), lambda i: (0, 0)),
            pl.BlockSpec((1, dout), lambda i: (0, 0)),
        ],
        out_specs=pl.BlockSpec((R, dout), lambda i: (i, 0)),
        out_shape=jax.ShapeDtypeStruct((n, dout), jnp.float32),
    )(feat, w, b.reshape(1, -1))


def _t1_body(hp_ref, w_ref, b_ref, o_ref):
    hp = hp_ref[0] + hp_ref[1]
    den = jnp.maximum(hp[:, 128:129], 1.0)
    tf = hp[:, 0:128] / den
    o_ref[...] = (jnp.dot(tf, w_ref[...],
                          preferred_element_type=jnp.float32) + b_ref[...])


def _t1(hpw, w, b):
    return pl.pallas_call(
        _t1_body,
        grid=(NUM_ITEM // R,),
        in_specs=[
            pl.BlockSpec((2, R, 136), lambda i: (0, i, 0)),
            pl.BlockSpec((128, 128), lambda i: (0, 0)),
            pl.BlockSpec((1, 128), lambda i: (0, 0)),
        ],
        out_specs=pl.BlockSpec((R, 128), lambda i: (i, 0)),
        out_shape=jax.ShapeDtypeStruct((NUM_ITEM, 128), jnp.float32),
    )(hpw, w, b.reshape(1, -1))


def _a2_body(x0_ref, cw_ref, x_ref, *m_refs):
    x0 = x0_ref[...]
    nrm = jnp.sqrt(jnp.sum(x0 * x0, axis=1, keepdims=True))
    x = x0 / jnp.maximum(nrm, 1e-12)
    x_ref[...] = x
    m = jnp.dot(x, cw_ref[...], preferred_element_type=jnp.float32)
    for cc in range(8):
        m_refs[cc][...] = m[:, CW * cc:CW * cc + CW]


def _a2(x0, conv1_w):
    outs = pl.pallas_call(
        _a2_body,
        grid=(NUM_NODES // R,),
        in_specs=[
            pl.BlockSpec((R, 128), lambda i: (i, 0)),
            pl.BlockSpec((128, 128), lambda i: (0, 0)),
        ],
        out_specs=[pl.BlockSpec((R, 128), lambda i: (i, 0))]
        + [pl.BlockSpec((R, CW), lambda i: (i, 0))] * 8,
        out_shape=[jax.ShapeDtypeStruct((NUM_NODES, 128), jnp.float32)]
        + [jax.ShapeDtypeStruct((NUM_NODES, CW), jnp.float32)] * 8,
    )(x0, conv1_w)
    return outs[0], outs[1:]


def _b_body(hpu_ref, hpi_ref, x_ref, id_ref, l1w_ref, l1b_ref, g1w_ref,
            g1b_ref, c2w_ref, x2_ref, *m_refs):
    user_blk = pl.program_id(0) < NUM_USER // R
    hp0 = jnp.where(user_blk, hpu_ref[0], hpi_ref[0])
    hp1 = jnp.where(user_blk, hpu_ref[1], hpi_ref[1])
    h = _lrelu(hp0 + hp1)
    hg = jnp.dot(h, g1w_ref[...], preferred_element_type=jnp.float32)
    xh = _lrelu(jnp.dot(x_ref[...], l1w_ref[...],
                        preferred_element_type=jnp.float32)
                + l1b_ref[...]) + id_ref[...]
    x2 = _lrelu(hg + g1b_ref[...] + xh)
    x2_ref[...] = x2
    m = jnp.dot(x2, c2w_ref[...], preferred_element_type=jnp.float32)
    for cc in range(4):
        m_refs[cc][...] = m[:, CW * cc:CW * cc + CW]


def _b_stage(hp, x, id_emb, p):
    hp_u, hp_i = hp
    nu = NUM_USER // R
    outs = pl.pallas_call(
        _b_body,
        grid=(NUM_NODES // R,),
        in_specs=[
            pl.BlockSpec((2, R, 128), lambda i: (0, jnp.minimum(i, nu - 1), 0)),
            pl.BlockSpec((2, R, 128), lambda i: (0, jnp.maximum(i - nu, 0), 0)),
            pl.BlockSpec((R, 128), lambda i: (i, 0)),
            pl.BlockSpec((R, 64), lambda i: (i, 0)),
            pl.BlockSpec((128, 64), lambda i: (0, 0)),
            pl.BlockSpec((1, 64), lambda i: (0, 0)),
            pl.BlockSpec((128, 64), lambda i: (0, 0)),
            pl.BlockSpec((1, 64), lambda i: (0, 0)),
            pl.BlockSpec((64, 64), lambda i: (0, 0)),
        ],
        out_specs=[pl.BlockSpec((R, 64), lambda i: (i, 0))]
        + [pl.BlockSpec((R, CW), lambda i: (i, 0))] * 4,
        out_shape=[jax.ShapeDtypeStruct((NUM_NODES, 64), jnp.float32)]
        + [jax.ShapeDtypeStruct((NUM_NODES, CW), jnp.float32)] * 4,
    )(hp_u, hp_i, x, id_emb, p['lin1_w'], p['lin1_b'].reshape(1, -1),
      p['g1_w'], p['g1_b'].reshape(1, -1), p['conv2_w'])
    return outs[0], outs[1:]


def _c_body(hpu_ref, hpi_ref, x2_ref, id_ref, l2w_ref, l2b_ref, g2w_ref,
            g2b_ref, rep_ref):
    user_blk = pl.program_id(0) < NUM_USER // R
    hp0 = jnp.where(user_blk, hpu_ref[0], hpi_ref[0])
    hp1 = jnp.where(user_blk, hpu_ref[1], hpi_ref[1])
    h = _lrelu(hp0 + hp1)
    hg = jnp.dot(h, g2w_ref[...], preferred_element_type=jnp.float32)
    xh = _lrelu(jnp.dot(x2_ref[...], l2w_ref[...],
                        preferred_element_type=jnp.float32)
                + l2b_ref[...]) + id_ref[...]
    rep_ref[...] = _lrelu(hg + g2b_ref[...] + xh)


def _c_stage(hp, x2, id_emb, p):
    hp_u, hp_i = hp
    nu = NUM_USER // R
    return pl.pallas_call(
        _c_body,
        grid=(NUM_NODES // R,),
        in_specs=[
            pl.BlockSpec((2, R, 64), lambda i: (0, jnp.minimum(i, nu - 1), 0)),
            pl.BlockSpec((2, R, 64), lambda i: (0, jnp.maximum(i - nu, 0), 0)),
            pl.BlockSpec((R, 64), lambda i: (i, 0)),
            pl.BlockSpec((R, 64), lambda i: (i, 0)),
            pl.BlockSpec((64, 64), lambda i: (0, 0)),
            pl.BlockSpec((1, 64), lambda i: (0, 0)),
            pl.BlockSpec((64, 64), lambda i: (0, 0)),
            pl.BlockSpec((1, 64), lambda i: (0, 0)),
        ],
        out_specs=pl.BlockSpec((R, 64), lambda i: (i, 0)),
        out_shape=jax.ShapeDtypeStruct((NUM_NODES, 64), jnp.float32),
    )(hp_u, hp_i, x2, id_emb, p['lin2_w'], p['lin2_b'].reshape(1, -1),
      p['g2_w'], p['g2_b'].reshape(1, -1))


def _score_body(g_ref, o_ref):
    gt_u, gt_p, gt_n = g_ref[0], g_ref[1], g_ref[2]
    gv_u, gv_p, gv_n = g_ref[3], g_ref[4], g_ref[5]
    ga_u, ga_p, ga_n = g_ref[6], g_ref[7], g_ref[8]
    pre_pos = jnp.sum(gt_u * gt_p, axis=1)
    pre_neg = jnp.sum(gt_u * gt_n, axis=1)
    pu = (gt_u + gv_u + ga_u) / 3.0
    pp = (gt_p + gv_p + ga_p) / 3.0
    pn = (gt_n + gv_n + ga_n) / 3.0
    post_pos = jnp.sum(pu * pp, axis=1)
    post_neg = jnp.sum(pu * pn, axis=1)
    o_ref[0, :] = post_pos * (1.0 / (1.0 + jnp.exp(-pre_pos)))
    o_ref[1, :] = post_neg * (1.0 / (1.0 + jnp.exp(-pre_neg)))
    o_ref[2, :] = pre_pos
    o_ref[3, :] = pre_neg


def _pad_idx(idx, pad_val, total, spread=1):
    pads = pad_val + (jnp.arange(total, dtype=jnp.int32) % spread)
    return lax.dynamic_update_slice(pads, idx.astype(jnp.int32), (0,))


def _pad_tab(t, rows):
    return jnp.pad(t, ((0, rows - t.shape[0]), (0, 0)))


def kernel(v_feat, a_feat, words_tensor, edge_index, id_embedding, word_emb,
           v_params, a_params, t_params, user_nodes, pos_item_nodes,
           neg_item_nodes):
    E = edge_index.shape[1]
    NI = E // 2
    W = words_tensor.shape[1]
    unit = NW * BLK
    nb_e = 2 * -(-NI // (2 * unit))
    nb_w = 2 * -(-W // (2 * unit))
    e_pad = nb_e * unit + BLK
    w_pad = nb_w * unit + BLK

    src, dst = edge_index[0], edge_index[1]
    src_u = _pad_idx(src[:NI], 0, e_pad, 8192)
    dst_i = _pad_idx(dst[:NI] - NUM_USER, NUM_ITEM, e_pad, 64)
    src_i = _pad_idx(src[NI:] - NUM_USER, 0, e_pad, 8192)
    dst_u = _pad_idx(dst[NI:], NUM_USER, e_pad, 112)
    wgat = _pad_idx(words_tensor[1], 0, w_pad, 8192)
    wsct = _pad_idx(words_tensor[0], NUM_ITEM, w_pad, 64)

    zeros16 = jnp.zeros((I_ROWS, CW), jnp.float32)
    zeros8 = jnp.zeros((I_ROWS, 8), jnp.float32)

    seg_ui8 = _make_seg(8, nb_e, I_ROWS, U_ROWS, CW)
    seg_iu8 = _make_seg(8, nb_e, U_ROWS, I_ROWS, CW)
    seg_ui4 = _make_seg(4, nb_e, I_ROWS, U_ROWS, CW)
    seg_iu4 = _make_seg(4, nb_e, U_ROWS, I_ROWS, CW)
    seg_w = _make_seg(17, nb_w, I_ROWS, W_ROWS, 8)

    def seg_edges(m_chunks, seg_ui, seg_iu):
        tabs_u = [_pad_tab(mc[:NUM_USER], U_ROWS) for mc in m_chunks]
        tabs_i = [_pad_tab(mc[NUM_USER:], I_ROWS) for mc in m_chunks]
        hp_i = seg_ui(*tabs_u, src_u, dst_i, zeros16)
        hp_u = seg_iu(*tabs_i, src_i, dst_u, zeros16)
        return hp_u, hp_i

    def gcn(p, temp):
        x0 = jnp.concatenate([p['preference'], temp], axis=0)
        x, m1c = _a2(x0, p['conv1_w'])
        hp1 = seg_edges(m1c, seg_ui8, seg_iu8)
        x2, m2c = _b_stage(hp1, x, id_embedding, p)
        hp2 = seg_edges(m2c, seg_ui4, seg_iu4)
        return _c_stage(hp2, x2, id_embedding, p)

    temp_v = _mm_bias(v_feat, v_params['mlp_w'], v_params['mlp_b'])
    temp_a = _mm_bias(a_feat, a_params['mlp_w'], a_params['mlp_b'])
    rep_v = gcn(v_params, temp_v)
    rep_a = gcn(a_params, temp_a)

    wtabs = [_pad_tab(word_emb[:, 8 * cc:8 * cc + 8], W_ROWS)
             for cc in range(16)]
    ones_tab = jnp.zeros((W_ROWS, 8), jnp.float32).at[:, 0].set(1.0)
    hpw = seg_w(*wtabs, ones_tab, wgat, wsct, zeros8)
    temp_t = _t1(hpw, t_params['mlp_w'], t_params['mlp_b'])
    rep_t = gcn(t_params, temp_t)

    g9 = _make_gather9()(
        rep_t, rep_v, rep_a,
        user_nodes.astype(jnp.int32), pos_item_nodes.astype(jnp.int32),
        neg_item_nodes.astype(jnp.int32))
    o = pl.pallas_call(
        _score_body,
        out_shape=jax.ShapeDtypeStruct((4, 1024), jnp.float32),
    )(g9)
    return (o[0], o[1], o[2], o[3])
